# baseline (speedup 1.0000x reference)
import functools

import jax
import jax.numpy as jnp
from jax import lax
from jax.experimental import pallas as pl
from jax.experimental.pallas import tpu as pltpu
from jax.experimental.pallas import tpu_sc as plsc

N = 10000
E = 320000
D = 128
HEADS = 8
HD = D // HEADS

NC = 2
NS = 16
NW = NC * NS
WIN = 64
NWIN = 79
EPH = NW * NWIN * WIN
EP = 2 * EPH
ACC_R = 10240
NBUF = 2
DUMMY = 10100
RPT = ACC_R // NS

@functools.lru_cache(maxsize=None)
def _mesh():
    return plsc.VectorSubcoreMesh(core_axis_name="c", subcore_axis_name="s")


def _celu2(x):
    return jnp.where(x > 0, x, 2.0 * (jnp.exp(0.5 * x) - 1.0))


def _fill2d(ref, rows, cols, value):
    vec = jnp.full((16,), value, jnp.float32)

    def body(i, _):
        for k in range(cols // 16):
            ref[i, pl.ds(16 * k, 16)] = vec
        return 0

    lax.fori_loop(0, rows, body, 0)


CW = 128
NCH = 6 * D // CW


@functools.lru_cache(maxsize=None)
def _build_sc_apply(nch):
    @functools.partial(
        pl.kernel,
        out_type=jax.ShapeDtypeStruct((NC, nch * D // CW, ACC_R, CW), jnp.float32),
        mesh=_mesh(),
        scratch_types=[
            pltpu.VMEM((NWIN, WIN), jnp.int32),
            pltpu.VMEM((NWIN, WIN), jnp.int32),
            pltpu.VMEM((NBUF, WIN, D), jnp.float32),
            pltpu.VMEM((WIN, D), jnp.float32),
            pltpu.VMEM_SHARED((ACC_R, D), jnp.float32),
            pltpu.SemaphoreType.DMA((NBUF,)),
        ],
    )
    def body(table_hbm, src_hbm, dst_hbm, out_hbm,
             src_v, dst_v, bufn, zero_v, acc, sems):
        cid = lax.axis_index("c")
        sid = lax.axis_index("s")
        wid = sid * NC + cid
        _fill2d(zero_v, WIN, D, 0.0)
        pltpu.sync_copy(src_hbm.at[wid], src_v)
        pltpu.sync_copy(dst_hbm.at[wid], dst_v)
        for ch6 in range(nch):
            tbl = table_hbm.at[ch6]
            for j in range(ACC_R // NS // WIN):
                pltpu.sync_copy(zero_v, acc.at[pl.ds(sid * (ACC_R // NS) + j * WIN, WIN)])
            plsc.subcore_barrier()

            def step(t, _):
                m = lax.rem(t, NBUF)

                @pl.when(t >= NBUF)
                def _drain():
                    w = t - NBUF
                    pltpu.make_async_copy(tbl.at[src_v.at[w]], bufn.at[m],
                                          sems.at[m]).wait()
                    pltpu.sync_copy(bufn.at[m], acc.at[dst_v.at[w]], add=True)

                @pl.when(t < NWIN)
                def _fire():
                    pltpu.async_copy(tbl.at[src_v.at[t]], bufn.at[m], sems.at[m])

                return 0

            lax.fori_loop(0, NWIN + NBUF, step, 0)
            plsc.subcore_barrier()
            pltpu.sync_copy(
                acc.at[pl.ds(sid * RPT, RPT)],
                out_hbm.at[cid, ch6, pl.ds(sid * RPT, RPT)],
            )
            plsc.subcore_barrier()

    return body


def _sc_apply(table, src, dst):
    nch = table.shape[0]
    k = _build_sc_apply(nch)
    ys = [k(table, src[h], dst[h]) for h in range(2)]
    y = jnp.concatenate(ys, axis=0)
    if CW != D:
        y = y.reshape(2 * NC, nch, D // CW, ACC_R, CW).transpose(0, 1, 3, 2, 4).reshape(2 * NC, nch, ACC_R, D)
    return y


GWIN = 128


@functools.lru_cache(maxsize=None)
def _build_sc_gather():
    @functools.partial(
        pl.kernel,
        out_type=jax.ShapeDtypeStruct((NW * 2 * GWIN, D), jnp.float32),
        mesh=_mesh(),
        scratch_types=[
            pltpu.VMEM((2, GWIN), jnp.int32),
            pltpu.VMEM((GWIN, D), jnp.float32),
            pltpu.SemaphoreType.DMA,
        ],
    )
    def body(table_hbm, idx_hbm, out_hbm, idx_v, rows_v, sem):
        cid = lax.axis_index("c")
        sid = lax.axis_index("s")
        wid = sid * NC + cid
        pltpu.sync_copy(idx_hbm.at[wid], idx_v)
        for w in range(2):
            pltpu.async_copy(table_hbm.at[idx_v.at[w]], rows_v, sem).wait()
            pltpu.sync_copy(rows_v, out_hbm.at[pl.ds((wid * 2 + w) * GWIN, GWIN)])

    return body


def _sc_gather(table, gidx):
    return _build_sc_gather()(table, gidx)


_TM = 2000


def _colsum_body(square, g, x_ref, o_ref):
    @pl.when((pl.program_id(0) == 0) & (pl.program_id(1) == 0))
    def _():
        o_ref[...] = jnp.zeros_like(o_ref)

    v = x_ref[0]
    if square:
        v = v * v
    colsum = jnp.sum(v, axis=0)
    row = lax.broadcasted_iota(jnp.int32, (g, D), 0)
    o_ref[...] += jnp.where(row == pl.program_id(0), colsum[None, :], 0.0)


def _tc_colsum(x, square):
    g = x.shape[0]
    return pl.pallas_call(
        functools.partial(_colsum_body, square, g),
        grid=(g, N // _TM),
        in_specs=[pl.BlockSpec((1, _TM, D), lambda c, i: (c, i, 0))],
        out_specs=pl.BlockSpec((g, D), lambda c, i: (0, 0)),
        out_shape=jax.ShapeDtypeStruct((g, D), jnp.float32),
    )(x)


def _mm1_body(x_ref, w_ref, dis_ref, o_ref):
    acc = jnp.dot(x_ref[...], w_ref[...], preferred_element_type=jnp.float32,
                  precision=lax.Precision.HIGHEST)
    o_ref[0] = acc * dis_ref[...]


def _tc_stage1_mm(xcat, wbig, dis):
    return pl.pallas_call(
        _mm1_body,
        grid=(6, N // _TM),
        in_specs=[
            pl.BlockSpec((_TM, 2 * D), lambda c, i: (i, 0)),
            pl.BlockSpec((2 * D, D), lambda c, i: (0, c)),
            pl.BlockSpec((_TM, 1), lambda c, i: (i, 0)),
        ],
        out_specs=pl.BlockSpec((1, _TM, D), lambda c, i: (c, i, 0)),
        out_shape=jax.ShapeDtypeStruct((6, N, D), jnp.float32),
    )(xcat, wbig, dis)


def _mmb_body(scale, act, x_ref, w_ref, b_ref, dis_ref, o_ref):
    acc = jnp.dot(x_ref[0], w_ref[0], preferred_element_type=jnp.float32,
                  precision=lax.Precision.HIGHEST)
    if b_ref is not None:
        acc = acc + b_ref[...]
    if scale:
        acc = acc * dis_ref[...]
    if act:
        acc = _celu2(acc)
    o_ref[0] = acc


def _tc_batched_mm(x, w, bias=None, dis=None, act=False):
    c = x.shape[0]
    wshare = w.shape[0] == 1
    specs = [
        pl.BlockSpec((1, _TM, D), lambda ci, i: (ci, i, 0)),
        pl.BlockSpec((1, D, D), (lambda ci, i: (0, 0, 0)) if wshare else (lambda ci, i: (ci, 0, 0))),
    ]
    args = [x, w]
    if bias is not None:
        bshare = bias.shape[0] == 1
        specs.append(pl.BlockSpec((1, D), (lambda ci, i: (0, 0)) if bshare else (lambda ci, i: (ci, 0))))
        args.append(bias)
    if dis is not None:
        specs.append(pl.BlockSpec((_TM, 1), lambda ci, i: (i, 0)))
        args.append(dis)

    def body(*refs):
        x_ref, w_ref = refs[0], refs[1]
        k = 2
        b_ref = None
        d_ref = None
        if bias is not None:
            b_ref = refs[k]
            k += 1
        if dis is not None:
            d_ref = refs[k]
            k += 1
        o_ref = refs[-1]
        _mmb_body(dis is not None, act, x_ref, w_ref, b_ref, d_ref, o_ref)

    return pl.pallas_call(
        body,
        grid=(c, N // _TM),
        in_specs=specs,
        out_specs=pl.BlockSpec((1, _TM, D), lambda ci, i: (ci, i, 0)),
        out_shape=jax.ShapeDtypeStruct((c, N, D), jnp.float32),
    )(*args)


def _dtf_v(x, a_ref, w_ref, b_ref):
    return jnp.tanh(a_ref[...] * x) * w_ref[...] + b_ref[...]


def _mid_body(y0_ref, y1_ref, y2_ref, y3_ref, t_ref, dis_ref, b1_ref, a_ref, w_ref, bb_ref, o_ref):
    ysum = y0_ref[0, 0] + y1_ref[0, 0] + y2_ref[0, 0] + y3_ref[0, 0]
    y = (ysum + t_ref[0]) * dis_ref[...] + b1_ref[0]
    va = _dtf_v(_celu2(y), a_ref.at[0], w_ref.at[0], bb_ref.at[0])
    vb = _dtf_v(y, a_ref.at[0], w_ref.at[0], bb_ref.at[0])
    even = (pl.program_id(0) % 2) == 0
    o_ref[0] = jnp.where(even, va, vb)


def _tc_mid(yparts, table, dis, b1, n1a, n1w, n1b):
    pspec = pl.BlockSpec((1, 1, D), lambda c, i: (c, 0, 0))
    return pl.pallas_call(
        _mid_body,
        grid=(6, N // _TM),
        in_specs=[
            pl.BlockSpec((1, 1, _TM, D), lambda c, i: (0, c, i, 0)),
            pl.BlockSpec((1, 1, _TM, D), lambda c, i: (1, c, i, 0)),
            pl.BlockSpec((1, 1, _TM, D), lambda c, i: (2, c, i, 0)),
            pl.BlockSpec((1, 1, _TM, D), lambda c, i: (3, c, i, 0)),
            pl.BlockSpec((1, _TM, D), lambda c, i: (c, i, 0)),
            pl.BlockSpec((_TM, 1), lambda c, i: (i, 0)),
            pspec, pspec, pspec, pspec,
        ],
        out_specs=pl.BlockSpec((1, _TM, D), lambda c, i: (c, i, 0)),
        out_shape=jax.ShapeDtypeStruct((6, N, D), jnp.float32),
    )(yparts, yparts, yparts, yparts, table, dis, b1[:, None, :], n1a[:, None, :],
      n1w[:, None, :], n1b[:, None, :])


def _fin_body(y0_ref, y1_ref, y2_ref, y3_ref, t_ref, dis_ref, b2_ref, a_ref, w_ref, bb_ref, o_ref):
    dis = dis_ref[...]
    b2 = b2_ref[0]
    ya = (y0_ref[0, 0] + y1_ref[0, 0] + y2_ref[0, 0] + y3_ref[0, 0] + t_ref[0]) * dis + b2
    yb = (y0_ref[0, 1] + y1_ref[0, 1] + y2_ref[0, 1] + y3_ref[0, 1] + t_ref[1]) * dis + b2
    x1aa = _dtf_v(_celu2(ya), a_ref.at[0], w_ref.at[0], bb_ref.at[0])
    o_ref[0] = x1aa * jax.nn.sigmoid(yb)


def _tc_fin(yparts, table, dis, b2, n2a, n2w, n2b):
    pspec = pl.BlockSpec((1, 1, D), lambda p, i: (p, 0, 0))
    return pl.pallas_call(
        _fin_body,
        grid=(3, N // _TM),
        in_specs=[
            pl.BlockSpec((1, 2, _TM, D), lambda p, i: (0, p, i, 0)),
            pl.BlockSpec((1, 2, _TM, D), lambda p, i: (1, p, i, 0)),
            pl.BlockSpec((1, 2, _TM, D), lambda p, i: (2, p, i, 0)),
            pl.BlockSpec((1, 2, _TM, D), lambda p, i: (3, p, i, 0)),
            pl.BlockSpec((2, _TM, D), lambda p, i: (p, i, 0)),
            pl.BlockSpec((_TM, 1), lambda p, i: (i, 0)),
            pspec, pspec, pspec, pspec,
        ],
        out_specs=pl.BlockSpec((1, _TM, D), lambda p, i: (p, i, 0)),
        out_shape=jax.ShapeDtypeStruct((3, N, D), jnp.float32),
    )(yparts, yparts, yparts, yparts, table, dis, b2[:, None, :], n2a[:, None, :],
      n2w[:, None, :], n2b[:, None, :])


def _disc_body(h_ref, u_ref, db_ref, o_ref):
    cols = []
    for b in range(4):
        cols.append(jnp.sum(h_ref[b] * u_ref[b][None, :], axis=1))
    s = jnp.stack(cols, axis=1) + db_ref[0, 0]
    o_ref[...] = _celu2(s)


def _tc_disc(hloc, u4, disc_b):
    return pl.pallas_call(
        _disc_body,
        grid=(N // _TM,),
        in_specs=[
            pl.BlockSpec((4, _TM, D), lambda i: (0, i, 0)),
            pl.BlockSpec((4, D), lambda i: (0, 0)),
            pl.BlockSpec((1, 1), lambda i: (0, 0)),
        ],
        out_specs=pl.BlockSpec((_TM, 4), lambda i: (i, 0)),
        out_shape=jax.ShapeDtypeStruct((N, 4), jnp.float32),
    )(hloc, u4, disc_b)


_TMA = 1000


def _dfam_body(h1_ref, h2_ref, h5_ref, h6_ref,
               qw_ref, qb_ref, kw_ref, kb_ref, vw_ref, vb_ref,
               a1w_ref, a1b_ref, ada_ref, adw_ref, adb_ref, a2w_ref, o_ref):
    zs = [h1_ref[...], h2_ref[...], (h5_ref[...] + h6_ref[...]) * 0.5]
    qw, qb = qw_ref[...], qb_ref[...]
    kw, kb = kw_ref[...], kb_ref[...]
    vw, vb = vw_ref[...], vb_ref[...]
    Q = [jnp.dot(z, qw, preferred_element_type=jnp.float32,
                  precision=lax.Precision.HIGHEST) + qb for z in zs]
    K = [jnp.dot(z, kw, preferred_element_type=jnp.float32,
                  precision=lax.Precision.HIGHEST) + kb for z in zs]
    V = [jnp.dot(z, vw, preferred_element_type=jnp.float32,
                  precision=lax.Precision.HIGHEST) + vb for z in zs]
    rows = lax.broadcasted_iota(jnp.int32, (D, HEADS), 0)
    cols = lax.broadcasted_iota(jnp.int32, (D, HEADS), 1)
    M = (rows // HD == cols).astype(jnp.float32)
    scale = float(HD) ** -0.5
    cos = []
    for i in range(3):
        s = [jnp.dot(Q[i] * K[j], M, preferred_element_type=jnp.float32,
                  precision=lax.Precision.HIGHEST) * scale
             for j in range(3)]
        m = jnp.maximum(jnp.maximum(s[0], s[1]), s[2])
        e = [jnp.exp(x - m) for x in s]
        den = e[0] + e[1] + e[2]
        o = sum(
            jnp.dot(e[j] / den, M.T, preferred_element_type=jnp.float32,
                  precision=lax.Precision.HIGHEST) * V[j]
            for j in range(3)
        )
        cos.append(_celu2(o))
    a1w, a1b = a1w_ref[...], a1b_ref[...]
    ada, adw, adb = ada_ref[...], adw_ref[...], adb_ref[...]
    a2w = a2w_ref[...]
    w = []
    for i in range(3):
        t = jnp.tanh(ada * (jnp.dot(cos[i], a1w, preferred_element_type=jnp.float32,
                  precision=lax.Precision.HIGHEST) + a1b)) * adw + adb
        w.append(jnp.sum(t * a2w, axis=1))
    wm = jnp.maximum(jnp.maximum(w[0], w[1]), w[2])
    ew = [jnp.exp(x - wm) for x in w]
    dw = ew[0] + ew[1] + ew[2]
    o_ref[...] = sum((ew[i] / dw)[:, None] * cos[i] for i in range(3))


def _tc_dfam(h1, h2, h5, h6, ap):
    full = lambda r, c: pl.BlockSpec((r, c), lambda i: (0, 0))
    tile = pl.BlockSpec((_TMA, D), lambda i: (i, 0))
    return pl.pallas_call(
        _dfam_body,
        grid=(N // _TMA,),
        in_specs=[tile, tile, tile, tile,
                  full(D, D), full(1, D), full(D, D), full(1, D),
                  full(D, D), full(1, D),
                  full(D, D), full(1, D), full(1, D), full(1, D), full(1, D),
                  full(1, D)],
        out_specs=tile,
        out_shape=jax.ShapeDtypeStruct((N, D), jnp.float32),
    )(h1, h2, h5, h6,
      ap['q_W'], ap['q_b'][None, :], ap['k_W'], ap['k_b'][None, :],
      ap['v_W'], ap['v_b'][None, :],
      ap['agg1_W'], ap['agg1_b'][None, :],
      jnp.broadcast_to(ap['agg_dtf']['alpha'], (D,))[None, :],
      ap['agg_dtf']['weight'][None, :], ap['agg_dtf']['bias'][None, :],
      ap['agg2_W'][:, 0][None, :])


_TMD = 512
DEC1 = 512
B = 4096


def _dec_body(e1_ref, e2_ref, w4_ref, b1_ref, w2_ref, b2_ref, o_ref):
    e1, e2 = e1_ref[...], e2_ref[...]
    parts = [e1 + e2, e1 * e2, e1, e2]
    acc = b1_ref[...]
    acc = acc + sum(
        jnp.dot(parts[k], w4_ref[k], preferred_element_type=jnp.float32,
                  precision=lax.Precision.HIGHEST)
        for k in range(4)
    )
    l1 = _celu2(acc)
    o_ref[...] = jnp.sum(l1 * w2_ref[...], axis=1, keepdims=True) + b2_ref[0, 0]


def _tc_dec(e1, e2, w4, b1, w2row, b2):
    tile = pl.BlockSpec((_TMD, D), lambda i: (i, 0))
    full = lambda *s: pl.BlockSpec(s, lambda i: tuple(0 for _ in s))
    return pl.pallas_call(
        _dec_body,
        grid=(B // _TMD,),
        in_specs=[tile, tile, full(4, D, DEC1), full(1, DEC1), full(1, DEC1),
                  full(1, 1)],
        out_specs=pl.BlockSpec((_TMD, 1), lambda i: (i, 0)),
        out_shape=jax.ShapeDtypeStruct((B, 1), jnp.float32),
    )(e1, e2, w4, b1, w2row, b2)


def _dtf_host(x, p):
    return jnp.tanh(p['alpha'] * x) * p['weight'] + p['bias']


def _dgct_gate(ss, p, eps=1e-05):
    embedding = jnp.power(ss[None, :] + eps, 0.5) * p['alpha']
    gamma = _dtf_host(embedding, p['dyt_gamma'])
    norm = gamma / jnp.power(jnp.mean(embedding * embedding, axis=1, keepdims=True) + eps, 0.5)
    beta = _dtf_host(embedding, p['dyt_beta'])
    gate = 1.0 + jax.nn.celu(embedding * norm + beta, alpha=2.0)
    return gate[0]


def _pad_edges(v, fill):
    pad = jnp.full((EP - E,), fill, jnp.int32)
    return jnp.concatenate([v, pad]).reshape(2, NW, NWIN, WIN)


def kernel(x_s, edge_index_s, x_f, edge_index_f, idx, params):
    p = params
    f32 = jnp.float32

    src_s = _pad_edges(edge_index_s[0], 0)
    dst_s = _pad_edges(edge_index_s[1], DUMMY)
    src_f = _pad_edges(edge_index_f[0], 0)
    dst_f = _pad_edges(edge_index_f[1], DUMMY)

    ones_tab = jnp.ones((1, N, D), f32)
    zsrc = jnp.zeros((2, NW, NWIN, WIN), jnp.int32)
    dga = _sc_apply(ones_tab, zsrc, dst_s)
    dgb = _sc_apply(ones_tab, zsrc, dst_f)
    deg = 1.0 + jnp.stack([jnp.sum(dga[:, 0, :N, 0], axis=0),
                           jnp.sum(dgb[:, 0, :N, 0], axis=0)])
    dis_s = lax.rsqrt(deg[0])[:, None]
    dis_f = lax.rsqrt(deg[1])[:, None]

    ss = _tc_colsum(jnp.stack([x_s, x_f]), square=True)
    ss_s, ss_f = ss[0], ss[1]

    groups = {
        's': (dis_s, src_s, dst_s, [(0, ss_s, 'enc1'), (1, ss_f, 'enc1'), (0, ss_s, 'enc3')]),
        'f': (dis_f, src_f, dst_f, [(0, ss_s, 'enc2'), (1, ss_f, 'enc2'), (0, ss_s, 'enc3')]),
    }
    xcat = jnp.concatenate([x_s, x_f], axis=1)

    hs = {}
    for gname, (dis, src, dst, passes) in groups.items():
        wbig = jnp.zeros((2 * D, 6 * D), f32)
        b1 = []
        n1 = []
        w2 = []
        b2 = []
        n2 = []
        for pi, (xsel, ssx, ename) in enumerate(passes):
            ep = p[ename]
            for br, gct in ((0, 'gct1'), (1, 'gct2')):
                gate = _dgct_gate(ssx, ep[gct])
                w1eff = gate[:, None] * ep['gc1a_W']
                c = 2 * pi + br
                wbig = wbig.at[xsel * D:(xsel + 1) * D, c * D:(c + 1) * D].set(w1eff)
                b1.append(ep['gc1a_b'])
                n1.append(ep['norm1'])
            w2.append(ep['gc2_W'])
            b2.append(ep['gc2_b'])
            n2.append(ep['norm2'])
        b1 = jnp.stack(b1)
        n1a = jnp.stack([jnp.broadcast_to(q['alpha'], (D,)) for q in n1])
        n1w = jnp.stack([q['weight'] for q in n1])
        n1b = jnp.stack([q['bias'] for q in n1])
        b2 = jnp.stack(b2)
        n2a = jnp.stack([jnp.broadcast_to(q['alpha'], (D,)) for q in n2])
        n2w = jnp.stack([q['weight'] for q in n2])
        n2b = jnp.stack([q['bias'] for q in n2])
        w2 = jnp.stack(w2)
        w2 = jnp.repeat(w2, 2, axis=0)

        table1 = _tc_stage1_mm(xcat, wbig, dis)
        yp1 = _sc_apply(table1, src, dst)
        x1 = _tc_mid(yp1, table1, dis, b1, n1a, n1w, n1b)
        table2 = _tc_batched_mm(x1, w2, dis=dis)
        yp2 = _sc_apply(table2, src, dst)
        hs[gname] = _tc_fin(yp2, table2, dis, b2, n2a, n2w, n2b)

    h1, h3, h5 = hs['s'][0], hs['s'][1], hs['s'][2]
    h2, h4, h6 = hs['f'][0], hs['f'][1], hs['f'][2]

    h4stack = jnp.stack([h1, h2, h3, h4])
    hloc = _tc_batched_mm(h4stack, p['local_W'][None], bias=p['local_b'][None],
                          act=True)

    csum = _tc_colsum(hloc[0:2], square=False)
    c1 = jax.nn.sigmoid(csum[0] / N @ p['global_W'] + p['global_b'])
    c2 = jax.nn.sigmoid(csum[1] / N @ p['global_W'] + p['global_b'])
    u1 = p['disc_W'] @ c1
    u2 = p['disc_W'] @ c2
    u4 = jnp.stack([u1, u2, u1, u2])
    sc4 = _tc_disc(hloc, u4, p['disc_b'].reshape(1, 1))
    out = sc4.T.reshape(4 * N)

    agg = _tc_dfam(hloc[0], hloc[1], h5, h6, p['attn'])

    gidx = jnp.concatenate([idx[0], idx[1] + 386]).reshape(NW, 2, GWIN)
    e12 = _sc_gather(agg, gidx)
    e1 = e12[:B]
    e2 = e12[B:]
    w4 = p['dec1_W'].reshape(4, D, DEC1)
    log = _tc_dec(e1, e2, w4, p['dec1_b'][None, :],
                  p['dec2_W'][:, 0][None, :], p['dec2_b'].reshape(1, 1))
    return (out, log)


# speedup vs baseline: 4.0302x; 4.0302x over previous
import functools

import jax
import jax.numpy as jnp
from jax import lax
from jax.experimental import pallas as pl
from jax.experimental.pallas import tpu as pltpu
from jax.experimental.pallas import tpu_sc as plsc

N = 10000
E = 320000
D = 128
HEADS = 8
HD = D // HEADS

NC = 2
NS = 16
NW = NC * NS
WIN = 64
NWIN = 79
EPH = NW * NWIN * WIN
EP = 2 * EPH
ACC_R = 10240
NBUF = 2
DUMMY = 10100
RPT = ACC_R // NS

@functools.lru_cache(maxsize=None)
def _mesh():
    return plsc.VectorSubcoreMesh(core_axis_name="c", subcore_axis_name="s")


def _celu2(x):
    return jnp.where(x > 0, x, 2.0 * (jnp.exp(0.5 * x) - 1.0))


def _fill2d(ref, rows, cols, value):
    vec = jnp.full((16,), value, jnp.float32)

    def body(i, _):
        for k in range(cols // 16):
            ref[i, pl.ds(16 * k, 16)] = vec
        return 0

    lax.fori_loop(0, rows, body, 0)


@functools.lru_cache(maxsize=None)
def _build_sc_degree():
    @functools.partial(
        pl.kernel,
        out_type=jax.ShapeDtypeStruct((NC, ACC_R, D), jnp.float32),
        mesh=_mesh(),
        scratch_types=[
            pltpu.VMEM((2 * NWIN, WIN), jnp.int32),
            pltpu.VMEM((WIN, D), jnp.float32),
            pltpu.VMEM((WIN, D), jnp.float32),
            pltpu.VMEM_SHARED((ACC_R, D), jnp.float32),
        ],
    )
    def body(dst_hbm, out_hbm, dst_v, ones_v, zero_v, acc):
        cid = lax.axis_index("c")
        sid = lax.axis_index("s")
        wid = sid * NC + cid
        _fill2d(zero_v, WIN, D, 0.0)
        _fill2d(ones_v, WIN, D, 1.0)
        pltpu.sync_copy(dst_hbm.at[wid], dst_v)
        for j in range(ACC_R // NS // WIN):
            pltpu.sync_copy(zero_v, acc.at[pl.ds(sid * (ACC_R // NS) + j * WIN, WIN)])
        plsc.subcore_barrier()

        def win(w, _):
            pltpu.sync_copy(ones_v, acc.at[dst_v.at[w]], add=True)
            return 0

        lax.fori_loop(0, 2 * NWIN, win, 0)
        plsc.subcore_barrier()
        pltpu.sync_copy(
            acc.at[pl.ds(sid * RPT, RPT)],
            out_hbm.at[cid, pl.ds(sid * RPT, RPT)],
        )
        plsc.subcore_barrier()

    return body


def _sc_degree(dst_halves):
    full = jnp.concatenate([dst_halves[0], dst_halves[1]], axis=1)
    y = _build_sc_degree()(full)
    return y[0, :N, 0] + y[1, :N, 0]


CW = 128
NCH = 6 * D // CW


@functools.lru_cache(maxsize=None)
def _build_sc_apply(nch):
    @functools.partial(
        pl.kernel,
        out_type=jax.ShapeDtypeStruct((NC, nch * D // CW, ACC_R, CW), jnp.float32),
        mesh=_mesh(),
        scratch_types=[
            pltpu.VMEM((NWIN, WIN), jnp.int32),
            pltpu.VMEM((NWIN, WIN), jnp.int32),
            pltpu.VMEM((NBUF, WIN, D), jnp.float32),
            pltpu.VMEM((WIN, D), jnp.float32),
            pltpu.VMEM_SHARED((ACC_R, D), jnp.float32),
            pltpu.SemaphoreType.DMA((NBUF,)),
        ],
    )
    def body(table_hbm, src_hbm, dst_hbm, out_hbm,
             src_v, dst_v, bufn, zero_v, acc, sems):
        cid = lax.axis_index("c")
        sid = lax.axis_index("s")
        wid = sid * NC + cid
        _fill2d(zero_v, WIN, D, 0.0)
        pltpu.sync_copy(src_hbm.at[wid], src_v)
        pltpu.sync_copy(dst_hbm.at[wid], dst_v)
        for ch6 in range(nch):
            tbl = table_hbm.at[ch6]
            for j in range(ACC_R // NS // WIN):
                pltpu.sync_copy(zero_v, acc.at[pl.ds(sid * (ACC_R // NS) + j * WIN, WIN)])
            plsc.subcore_barrier()

            def step(t, _):
                m = lax.rem(t, NBUF)

                @pl.when(t >= NBUF)
                def _drain():
                    w = t - NBUF
                    pltpu.make_async_copy(tbl.at[src_v.at[w]], bufn.at[m],
                                          sems.at[m]).wait()
                    pltpu.sync_copy(bufn.at[m], acc.at[dst_v.at[w]], add=True)

                @pl.when(t < NWIN)
                def _fire():
                    pltpu.async_copy(tbl.at[src_v.at[t]], bufn.at[m], sems.at[m])

                return 0

            lax.fori_loop(0, NWIN + NBUF, step, 0)
            plsc.subcore_barrier()
            pltpu.sync_copy(
                acc.at[pl.ds(sid * RPT, RPT)],
                out_hbm.at[cid, ch6, pl.ds(sid * RPT, RPT)],
            )
            plsc.subcore_barrier()

    return body


def _sc_apply(table, src, dst):
    nch = table.shape[0]
    k = _build_sc_apply(nch)
    ys = [k(table, src[h], dst[h]) for h in range(2)]
    y = jnp.concatenate(ys, axis=0)
    if CW != D:
        y = y.reshape(2 * NC, nch, D // CW, ACC_R, CW).transpose(0, 1, 3, 2, 4).reshape(2 * NC, nch, ACC_R, D)
    return y


GWIN = 128


@functools.lru_cache(maxsize=None)
def _build_sc_gather():
    @functools.partial(
        pl.kernel,
        out_type=jax.ShapeDtypeStruct((NW * 2 * GWIN, D), jnp.float32),
        mesh=_mesh(),
        scratch_types=[
            pltpu.VMEM((2, GWIN), jnp.int32),
            pltpu.VMEM((GWIN, D), jnp.float32),
            pltpu.SemaphoreType.DMA,
        ],
    )
    def body(table_hbm, idx_hbm, out_hbm, idx_v, rows_v, sem):
        cid = lax.axis_index("c")
        sid = lax.axis_index("s")
        wid = sid * NC + cid
        pltpu.sync_copy(idx_hbm.at[wid], idx_v)
        for w in range(2):
            pltpu.async_copy(table_hbm.at[idx_v.at[w]], rows_v, sem).wait()
            pltpu.sync_copy(rows_v, out_hbm.at[pl.ds((wid * 2 + w) * GWIN, GWIN)])

    return body


def _sc_gather(table, gidx):
    return _build_sc_gather()(table, gidx)


_TM = 2000


def _colsum_body(square, g, x_ref, o_ref):
    @pl.when((pl.program_id(0) == 0) & (pl.program_id(1) == 0))
    def _():
        o_ref[...] = jnp.zeros_like(o_ref)

    v = x_ref[0]
    if square:
        v = v * v
    colsum = jnp.sum(v, axis=0)
    row = lax.broadcasted_iota(jnp.int32, (g, D), 0)
    o_ref[...] += jnp.where(row == pl.program_id(0), colsum[None, :], 0.0)


def _tc_colsum(x, square):
    g = x.shape[0]
    return pl.pallas_call(
        functools.partial(_colsum_body, square, g),
        grid=(g, N // _TM),
        in_specs=[pl.BlockSpec((1, _TM, D), lambda c, i: (c, i, 0))],
        out_specs=pl.BlockSpec((g, D), lambda c, i: (0, 0)),
        out_shape=jax.ShapeDtypeStruct((g, D), jnp.float32),
    )(x)


def _mm1_body(x_ref, w_ref, dis_ref, o_ref):
    acc = jnp.dot(x_ref[...], w_ref[...], preferred_element_type=jnp.float32,
                  precision=lax.Precision.HIGHEST)
    o_ref[0] = acc * dis_ref[...]


def _tc_stage1_mm(xcat, wbig, dis):
    return pl.pallas_call(
        _mm1_body,
        grid=(6, N // _TM),
        in_specs=[
            pl.BlockSpec((_TM, 2 * D), lambda c, i: (i, 0)),
            pl.BlockSpec((2 * D, D), lambda c, i: (0, c)),
            pl.BlockSpec((_TM, 1), lambda c, i: (i, 0)),
        ],
        out_specs=pl.BlockSpec((1, _TM, D), lambda c, i: (c, i, 0)),
        out_shape=jax.ShapeDtypeStruct((6, N, D), jnp.float32),
    )(xcat, wbig, dis)


def _mmb_body(scale, act, x_ref, w_ref, b_ref, dis_ref, o_ref):
    acc = jnp.dot(x_ref[0], w_ref[0], preferred_element_type=jnp.float32,
                  precision=lax.Precision.HIGHEST)
    if b_ref is not None:
        acc = acc + b_ref[...]
    if scale:
        acc = acc * dis_ref[...]
    if act:
        acc = _celu2(acc)
    o_ref[0] = acc


def _tc_batched_mm(x, w, bias=None, dis=None, act=False):
    c = x.shape[0]
    wshare = w.shape[0] == 1
    specs = [
        pl.BlockSpec((1, _TM, D), lambda ci, i: (ci, i, 0)),
        pl.BlockSpec((1, D, D), (lambda ci, i: (0, 0, 0)) if wshare else (lambda ci, i: (ci, 0, 0))),
    ]
    args = [x, w]
    if bias is not None:
        bshare = bias.shape[0] == 1
        specs.append(pl.BlockSpec((1, D), (lambda ci, i: (0, 0)) if bshare else (lambda ci, i: (ci, 0))))
        args.append(bias)
    if dis is not None:
        specs.append(pl.BlockSpec((_TM, 1), lambda ci, i: (i, 0)))
        args.append(dis)

    def body(*refs):
        x_ref, w_ref = refs[0], refs[1]
        k = 2
        b_ref = None
        d_ref = None
        if bias is not None:
            b_ref = refs[k]
            k += 1
        if dis is not None:
            d_ref = refs[k]
            k += 1
        o_ref = refs[-1]
        _mmb_body(dis is not None, act, x_ref, w_ref, b_ref, d_ref, o_ref)

    return pl.pallas_call(
        body,
        grid=(c, N // _TM),
        in_specs=specs,
        out_specs=pl.BlockSpec((1, _TM, D), lambda ci, i: (ci, i, 0)),
        out_shape=jax.ShapeDtypeStruct((c, N, D), jnp.float32),
    )(*args)


def _dtf_v(x, a_ref, w_ref, b_ref):
    return jnp.tanh(a_ref[...] * x) * w_ref[...] + b_ref[...]


def _mid_body(y0_ref, y1_ref, y2_ref, y3_ref, t_ref, dis_ref, b1_ref, a_ref, w_ref, bb_ref, o_ref):
    ysum = y0_ref[0, 0] + y1_ref[0, 0] + y2_ref[0, 0] + y3_ref[0, 0]
    y = (ysum + t_ref[0]) * dis_ref[...] + b1_ref[0]
    va = _dtf_v(_celu2(y), a_ref.at[0], w_ref.at[0], bb_ref.at[0])
    vb = _dtf_v(y, a_ref.at[0], w_ref.at[0], bb_ref.at[0])
    even = (pl.program_id(0) % 2) == 0
    o_ref[0] = jnp.where(even, va, vb)


def _tc_mid(yparts, table, dis, b1, n1a, n1w, n1b):
    pspec = pl.BlockSpec((1, 1, D), lambda c, i: (c, 0, 0))
    return pl.pallas_call(
        _mid_body,
        grid=(6, N // _TM),
        in_specs=[
            pl.BlockSpec((1, 1, _TM, D), lambda c, i: (0, c, i, 0)),
            pl.BlockSpec((1, 1, _TM, D), lambda c, i: (1, c, i, 0)),
            pl.BlockSpec((1, 1, _TM, D), lambda c, i: (2, c, i, 0)),
            pl.BlockSpec((1, 1, _TM, D), lambda c, i: (3, c, i, 0)),
            pl.BlockSpec((1, _TM, D), lambda c, i: (c, i, 0)),
            pl.BlockSpec((_TM, 1), lambda c, i: (i, 0)),
            pspec, pspec, pspec, pspec,
        ],
        out_specs=pl.BlockSpec((1, _TM, D), lambda c, i: (c, i, 0)),
        out_shape=jax.ShapeDtypeStruct((6, N, D), jnp.float32),
    )(yparts, yparts, yparts, yparts, table, dis, b1[:, None, :], n1a[:, None, :],
      n1w[:, None, :], n1b[:, None, :])


def _fin_body(y0_ref, y1_ref, y2_ref, y3_ref, t_ref, dis_ref, b2_ref, a_ref, w_ref, bb_ref, o_ref):
    dis = dis_ref[...]
    b2 = b2_ref[0]
    ya = (y0_ref[0, 0] + y1_ref[0, 0] + y2_ref[0, 0] + y3_ref[0, 0] + t_ref[0]) * dis + b2
    yb = (y0_ref[0, 1] + y1_ref[0, 1] + y2_ref[0, 1] + y3_ref[0, 1] + t_ref[1]) * dis + b2
    x1aa = _dtf_v(_celu2(ya), a_ref.at[0], w_ref.at[0], bb_ref.at[0])
    o_ref[0] = x1aa * jax.nn.sigmoid(yb)


def _tc_fin(yparts, table, dis, b2, n2a, n2w, n2b):
    pspec = pl.BlockSpec((1, 1, D), lambda p, i: (p, 0, 0))
    return pl.pallas_call(
        _fin_body,
        grid=(3, N // _TM),
        in_specs=[
            pl.BlockSpec((1, 2, _TM, D), lambda p, i: (0, p, i, 0)),
            pl.BlockSpec((1, 2, _TM, D), lambda p, i: (1, p, i, 0)),
            pl.BlockSpec((1, 2, _TM, D), lambda p, i: (2, p, i, 0)),
            pl.BlockSpec((1, 2, _TM, D), lambda p, i: (3, p, i, 0)),
            pl.BlockSpec((2, _TM, D), lambda p, i: (p, i, 0)),
            pl.BlockSpec((_TM, 1), lambda p, i: (i, 0)),
            pspec, pspec, pspec, pspec,
        ],
        out_specs=pl.BlockSpec((1, _TM, D), lambda p, i: (p, i, 0)),
        out_shape=jax.ShapeDtypeStruct((3, N, D), jnp.float32),
    )(yparts, yparts, yparts, yparts, table, dis, b2[:, None, :], n2a[:, None, :],
      n2w[:, None, :], n2b[:, None, :])


def _disc_body(h_ref, u_ref, db_ref, o_ref):
    cols = []
    for b in range(4):
        cols.append(jnp.sum(h_ref[b] * u_ref[b][None, :], axis=1))
    s = jnp.stack(cols, axis=1) + db_ref[0, 0]
    o_ref[...] = _celu2(s)


def _tc_disc(hloc, u4, disc_b):
    return pl.pallas_call(
        _disc_body,
        grid=(N // _TM,),
        in_specs=[
            pl.BlockSpec((4, _TM, D), lambda i: (0, i, 0)),
            pl.BlockSpec((4, D), lambda i: (0, 0)),
            pl.BlockSpec((1, 1), lambda i: (0, 0)),
        ],
        out_specs=pl.BlockSpec((_TM, 4), lambda i: (i, 0)),
        out_shape=jax.ShapeDtypeStruct((N, 4), jnp.float32),
    )(hloc, u4, disc_b)


_TMA = 1000


def _dfam_body(h1_ref, h2_ref, h5_ref, h6_ref,
               qw_ref, qb_ref, kw_ref, kb_ref, vw_ref, vb_ref,
               a1w_ref, a1b_ref, ada_ref, adw_ref, adb_ref, a2w_ref, o_ref):
    zs = [h1_ref[...], h2_ref[...], (h5_ref[...] + h6_ref[...]) * 0.5]
    qw, qb = qw_ref[...], qb_ref[...]
    kw, kb = kw_ref[...], kb_ref[...]
    vw, vb = vw_ref[...], vb_ref[...]
    Q = [jnp.dot(z, qw, preferred_element_type=jnp.float32,
                  precision=lax.Precision.HIGHEST) + qb for z in zs]
    K = [jnp.dot(z, kw, preferred_element_type=jnp.float32,
                  precision=lax.Precision.HIGHEST) + kb for z in zs]
    V = [jnp.dot(z, vw, preferred_element_type=jnp.float32,
                  precision=lax.Precision.HIGHEST) + vb for z in zs]
    rows = lax.broadcasted_iota(jnp.int32, (D, HEADS), 0)
    cols = lax.broadcasted_iota(jnp.int32, (D, HEADS), 1)
    M = (rows // HD == cols).astype(jnp.float32)
    scale = float(HD) ** -0.5
    cos = []
    for i in range(3):
        s = [jnp.dot(Q[i] * K[j], M, preferred_element_type=jnp.float32,
                  precision=lax.Precision.HIGHEST) * scale
             for j in range(3)]
        m = jnp.maximum(jnp.maximum(s[0], s[1]), s[2])
        e = [jnp.exp(x - m) for x in s]
        den = e[0] + e[1] + e[2]
        o = sum(
            jnp.dot(e[j] / den, M.T, preferred_element_type=jnp.float32,
                  precision=lax.Precision.HIGHEST) * V[j]
            for j in range(3)
        )
        cos.append(_celu2(o))
    a1w, a1b = a1w_ref[...], a1b_ref[...]
    ada, adw, adb = ada_ref[...], adw_ref[...], adb_ref[...]
    a2w = a2w_ref[...]
    w = []
    for i in range(3):
        t = jnp.tanh(ada * (jnp.dot(cos[i], a1w, preferred_element_type=jnp.float32,
                  precision=lax.Precision.HIGHEST) + a1b)) * adw + adb
        w.append(jnp.sum(t * a2w, axis=1))
    wm = jnp.maximum(jnp.maximum(w[0], w[1]), w[2])
    ew = [jnp.exp(x - wm) for x in w]
    dw = ew[0] + ew[1] + ew[2]
    o_ref[...] = sum((ew[i] / dw)[:, None] * cos[i] for i in range(3))


def _tc_dfam(h1, h2, h5, h6, ap):
    full = lambda r, c: pl.BlockSpec((r, c), lambda i: (0, 0))
    tile = pl.BlockSpec((_TMA, D), lambda i: (i, 0))
    return pl.pallas_call(
        _dfam_body,
        grid=(N // _TMA,),
        in_specs=[tile, tile, tile, tile,
                  full(D, D), full(1, D), full(D, D), full(1, D),
                  full(D, D), full(1, D),
                  full(D, D), full(1, D), full(1, D), full(1, D), full(1, D),
                  full(1, D)],
        out_specs=tile,
        out_shape=jax.ShapeDtypeStruct((N, D), jnp.float32),
    )(h1, h2, h5, h6,
      ap['q_W'], ap['q_b'][None, :], ap['k_W'], ap['k_b'][None, :],
      ap['v_W'], ap['v_b'][None, :],
      ap['agg1_W'], ap['agg1_b'][None, :],
      jnp.broadcast_to(ap['agg_dtf']['alpha'], (D,))[None, :],
      ap['agg_dtf']['weight'][None, :], ap['agg_dtf']['bias'][None, :],
      ap['agg2_W'][:, 0][None, :])


_TMD = 512
DEC1 = 512
B = 4096


def _dec_body(e1_ref, e2_ref, w4_ref, b1_ref, w2_ref, b2_ref, o_ref):
    e1, e2 = e1_ref[...], e2_ref[...]
    parts = [e1 + e2, e1 * e2, e1, e2]
    acc = b1_ref[...]
    acc = acc + sum(
        jnp.dot(parts[k], w4_ref[k], preferred_element_type=jnp.float32,
                  precision=lax.Precision.HIGHEST)
        for k in range(4)
    )
    l1 = _celu2(acc)
    o_ref[...] = jnp.sum(l1 * w2_ref[...], axis=1, keepdims=True) + b2_ref[0, 0]


def _tc_dec(e1, e2, w4, b1, w2row, b2):
    tile = pl.BlockSpec((_TMD, D), lambda i: (i, 0))
    full = lambda *s: pl.BlockSpec(s, lambda i: tuple(0 for _ in s))
    return pl.pallas_call(
        _dec_body,
        grid=(B // _TMD,),
        in_specs=[tile, tile, full(4, D, DEC1), full(1, DEC1), full(1, DEC1),
                  full(1, 1)],
        out_specs=pl.BlockSpec((_TMD, 1), lambda i: (i, 0)),
        out_shape=jax.ShapeDtypeStruct((B, 1), jnp.float32),
    )(e1, e2, w4, b1, w2row, b2)


def _dtf_host(x, p):
    return jnp.tanh(p['alpha'] * x) * p['weight'] + p['bias']


def _dgct_gate(ss, p, eps=1e-05):
    embedding = jnp.power(ss[None, :] + eps, 0.5) * p['alpha']
    gamma = _dtf_host(embedding, p['dyt_gamma'])
    norm = gamma / jnp.power(jnp.mean(embedding * embedding, axis=1, keepdims=True) + eps, 0.5)
    beta = _dtf_host(embedding, p['dyt_beta'])
    gate = 1.0 + jax.nn.celu(embedding * norm + beta, alpha=2.0)
    return gate[0]


def _pad_edges(v, fill):
    pad = jnp.full((EP - E,), fill, jnp.int32)
    return jnp.concatenate([v, pad]).reshape(2, NW, NWIN, WIN)


def kernel(x_s, edge_index_s, x_f, edge_index_f, idx, params):
    p = params
    f32 = jnp.float32

    src_s = _pad_edges(edge_index_s[0], 0)
    dst_s = _pad_edges(edge_index_s[1], DUMMY)
    src_f = _pad_edges(edge_index_f[0], 0)
    dst_f = _pad_edges(edge_index_f[1], DUMMY)

    deg = 1.0 + jnp.stack([_sc_degree(dst_s), _sc_degree(dst_f)])
    dis_s = lax.rsqrt(deg[0])[:, None]
    dis_f = lax.rsqrt(deg[1])[:, None]

    ss = _tc_colsum(jnp.stack([x_s, x_f]), square=True)
    ss_s, ss_f = ss[0], ss[1]

    groups = {
        's': (dis_s, src_s, dst_s, [(0, ss_s, 'enc1'), (1, ss_f, 'enc1'), (0, ss_s, 'enc3')]),
        'f': (dis_f, src_f, dst_f, [(0, ss_s, 'enc2'), (1, ss_f, 'enc2'), (0, ss_s, 'enc3')]),
    }
    xcat = jnp.concatenate([x_s, x_f], axis=1)

    hs = {}
    for gname, (dis, src, dst, passes) in groups.items():
        wbig = jnp.zeros((2 * D, 6 * D), f32)
        b1 = []
        n1 = []
        w2 = []
        b2 = []
        n2 = []
        for pi, (xsel, ssx, ename) in enumerate(passes):
            ep = p[ename]
            for br, gct in ((0, 'gct1'), (1, 'gct2')):
                gate = _dgct_gate(ssx, ep[gct])
                w1eff = gate[:, None] * ep['gc1a_W']
                c = 2 * pi + br
                wbig = wbig.at[xsel * D:(xsel + 1) * D, c * D:(c + 1) * D].set(w1eff)
                b1.append(ep['gc1a_b'])
                n1.append(ep['norm1'])
            w2.append(ep['gc2_W'])
            b2.append(ep['gc2_b'])
            n2.append(ep['norm2'])
        b1 = jnp.stack(b1)
        n1a = jnp.stack([jnp.broadcast_to(q['alpha'], (D,)) for q in n1])
        n1w = jnp.stack([q['weight'] for q in n1])
        n1b = jnp.stack([q['bias'] for q in n1])
        b2 = jnp.stack(b2)
        n2a = jnp.stack([jnp.broadcast_to(q['alpha'], (D,)) for q in n2])
        n2w = jnp.stack([q['weight'] for q in n2])
        n2b = jnp.stack([q['bias'] for q in n2])
        w2 = jnp.stack(w2)
        w2 = jnp.repeat(w2, 2, axis=0)

        table1 = _tc_stage1_mm(xcat, wbig, dis)
        yp1 = _sc_apply(table1, src, dst)
        x1 = _tc_mid(yp1, table1, dis, b1, n1a, n1w, n1b)
        table2 = _tc_batched_mm(x1, w2, dis=dis)
        yp2 = _sc_apply(table2, src, dst)
        hs[gname] = _tc_fin(yp2, table2, dis, b2, n2a, n2w, n2b)

    h1, h3, h5 = hs['s'][0], hs['s'][1], hs['s'][2]
    h2, h4, h6 = hs['f'][0], hs['f'][1], hs['f'][2]

    h4stack = jnp.stack([h1, h2, h3, h4])
    hloc = _tc_batched_mm(h4stack, p['local_W'][None], bias=p['local_b'][None],
                          act=True)

    csum = _tc_colsum(hloc[0:2], square=False)
    c1 = jax.nn.sigmoid(csum[0] / N @ p['global_W'] + p['global_b'])
    c2 = jax.nn.sigmoid(csum[1] / N @ p['global_W'] + p['global_b'])
    u1 = p['disc_W'] @ c1
    u2 = p['disc_W'] @ c2
    u4 = jnp.stack([u1, u2, u1, u2])
    sc4 = _tc_disc(hloc, u4, p['disc_b'].reshape(1, 1))
    out = sc4.T.reshape(4 * N)

    agg = _tc_dfam(hloc[0], hloc[1], h5, h6, p['attn'])

    gidx = jnp.concatenate([idx[0], idx[1] + 386]).reshape(NW, 2, GWIN)
    e12 = _sc_gather(agg, gidx)
    e1 = e12[:B]
    e2 = e12[B:]
    w4 = p['dec1_W'].reshape(4, D, DEC1)
    log = _tc_dec(e1, e2, w4, p['dec1_b'][None, :],
                  p['dec2_W'][:, 0][None, :], p['dec2_b'].reshape(1, 1))
    return (out, log)
